# Initial kernel scaffold; baseline (speedup 1.0000x reference)
import jax
import jax.numpy as jnp
from jax.experimental import pallas as pl


def kernel(dense_xyz, dense_feats, inv_neighbors, inv_k, inv_idx, nei_inds, u1_W, u1_g, u1_be, wn_W0, wn_b0, wn_g0, wn_be0, wn_W1, wn_b1, wn_g1, wn_be1, wn_W2, wn_b2, wn_g2, wn_be2, lin_W, lin_b, lin_g, lin_be, u2_W, u2_g, u2_be):
    raise NotImplementedError("write your pallas kernel here")


# speedup vs baseline: 8.6534x; 8.6534x over previous
import functools

import jax
import jax.numpy as jnp
from jax import lax
from jax.experimental import pallas as pl
from jax.experimental.pallas import tpu as pltpu
from jax.experimental.pallas import tpu_sc as plsc

N = 10000
K = 32
CIN = 128
C4 = 32
CMID = 16
CHUNK = 80
NCHUNK = (N * K) // CHUNK
NWORK = 32
CPW = NCHUNK // NWORK
GRP = 5
NGRP = CPW // GRP
P3 = 400
PK = P3 * K

_BN_INV = (1.0 + 1e-5) ** -0.5


def _bn(x, g, b):
    return g * (x * _BN_INV) + b


def _lrelu(x):
    return jnp.where(x > 0, x, 0.1 * x)


def _sc_gather_kernel(featsb_hbm, xyzt_hbm, nei_hbm, gf_hbm, wni_hbm,
                      idx_v, gf_bufs, wni_bufs, xyz_v,
                      gsem, osem):
    wid = lax.axis_index("s") * 2 + lax.axis_index("c")
    base = wid * CPW
    pltpu.sync_copy(nei_hbm.at[wid], idx_v)
    pltpu.sync_copy(xyzt_hbm, xyz_v)

    def wni_slot(b):
        return wni_bufs.at[pl.ds(b * CHUNK * 8, CHUNK * 8)]

    zero16 = jnp.zeros((16,), jnp.float32)
    for q in range(GRP * CHUNK * 8 // 16):
        wni_bufs[pl.ds(q * 16, 16)] = zero16

    lanes = lax.iota(jnp.int32, 16)

    def group(it, carry):
        handles = []
        for b in range(GRP):
            ci = it * GRP + b
            handles.append(
                pltpu.async_copy(featsb_hbm.at[idx_v.at[ci, 0]],
                                 gf_bufs.at[b], gsem))
        for b in range(GRP):
            ci = it * GRP + b
            ebase = (base + ci) * CHUNK
            for r in range(CHUNK // 16):
                iv = idx_v[ci, 0, pl.ds(r * 16, 16)]
                j = r * 16 + lanes
                p = lax.shift_right_logical(ebase + j, 5)
                col = b * CHUNK * 8 + 8 * j
                nx = plsc.load_gather(xyz_v, [iv])
                cx = plsc.load_gather(xyz_v, [p])
                plsc.store_scatter(wni_bufs, [col], nx - cx)
                ny = plsc.load_gather(xyz_v, [iv + N])
                cy = plsc.load_gather(xyz_v, [p + N])
                plsc.store_scatter(wni_bufs, [col + 1], ny - cy)
                nz = plsc.load_gather(xyz_v, [iv + 2 * N])
                cz = plsc.load_gather(xyz_v, [p + 2 * N])
                plsc.store_scatter(wni_bufs, [col + 2], nz - cz)
        for h in handles:
            h.wait()
        outs = []
        for b in range(GRP):
            ci = it * GRP + b
            outs.append(pltpu.async_copy(gf_bufs.at[b],
                                         gf_hbm.at[base + ci], osem))
            outs.append(pltpu.async_copy(
                wni_slot(b),
                wni_hbm.at[pl.ds((base + ci) * CHUNK * 8, CHUNK * 8)], osem))
        for h in outs:
            h.wait()
        return carry

    lax.fori_loop(0, NGRP, group, 0)


def _dense_kernel(gf_ref, loc_ref, feats_ref,
                  u1w_ref, u1g_ref, u1be_ref,
                  w0_ref, b0_ref, g0_ref, be0_ref,
                  w1_ref, b1_ref, g1_ref, be1_ref,
                  w2_ref, b2_ref, g2_ref, be2_ref,
                  lw_ref, lb_ref, lg_ref, lbe_ref,
                  u2w_ref, u2g_ref, u2be_ref,
                  nf_ref):
    fx = jnp.dot(gf_ref[...], u1w_ref[...],
                 preferred_element_type=jnp.float32)
    fx = _lrelu(_bn(fx, u1g_ref[...], u1be_ref[...]))

    locf = loc_ref[...]
    h = jnp.dot(locf, w0_ref[...], preferred_element_type=jnp.float32)
    h = jax.nn.relu(_bn(h + b0_ref[...], g0_ref[...], be0_ref[...]))
    h = jnp.dot(h, w1_ref[...], preferred_element_type=jnp.float32)
    h = jax.nn.relu(_bn(h + b1_ref[...], g1_ref[...], be1_ref[...]))
    h = jnp.dot(h, w2_ref[...], preferred_element_type=jnp.float32)
    w = jax.nn.relu(_bn(h + b2_ref[...], g2_ref[...], be2_ref[...]))

    gf3 = fx.reshape(P3, K, C4)
    w3 = w.reshape(P3, K, CMID)
    nf = lax.dot_general(gf3, w3, (((1,), (1,)), ((0,), (0,))),
                         preferred_element_type=jnp.float32)
    nf = nf.reshape(P3, C4 * CMID)

    y = jnp.dot(nf, lw_ref[...], preferred_element_type=jnp.float32)
    y = jax.nn.relu(_bn(y + lb_ref[...], lg_ref[...], lbe_ref[...]))
    y = jnp.dot(y, u2w_ref[...], preferred_element_type=jnp.float32)
    y = _bn(y, u2g_ref[...], u2be_ref[...])
    nf_ref[...] = _lrelu(y + feats_ref[...])


def kernel(dense_xyz, dense_feats, inv_neighbors, inv_k, inv_idx, nei_inds,
           u1_W, u1_g, u1_be,
           wn_W0, wn_b0, wn_g0, wn_be0,
           wn_W1, wn_b1, wn_g1, wn_be1,
           wn_W2, wn_b2, wn_g2, wn_be2,
           lin_W, lin_b, lin_g, lin_be,
           u2_W, u2_g, u2_be):
    feats = dense_feats[0]
    xyzt = jnp.transpose(dense_xyz[0]).reshape(3 * N)
    nei = nei_inds.reshape(NWORK, CPW, 1, CHUNK)

    gather = functools.partial(
        pl.kernel,
        mesh=plsc.VectorSubcoreMesh(core_axis_name="c", subcore_axis_name="s"),
        compiler_params=pltpu.CompilerParams(needs_layout_passes=False),
        out_type=[
            jax.ShapeDtypeStruct((NCHUNK, CHUNK, CIN), jnp.float32),
            jax.ShapeDtypeStruct((NCHUNK * CHUNK * 8,), jnp.float32),
        ],
        scratch_types=[
            pltpu.VMEM((CPW, 1, CHUNK), jnp.int32),
            pltpu.VMEM((GRP, CHUNK, CIN), jnp.float32),
            pltpu.VMEM((GRP * CHUNK * 8,), jnp.float32),
            pltpu.VMEM((3 * N,), jnp.float32),
            pltpu.SemaphoreType.DMA,
            pltpu.SemaphoreType.DMA,
        ],
    )(_sc_gather_kernel)
    gf, wni8 = gather(feats, xyzt, nei)
    gf = gf.reshape(N * K, CIN)
    wni8 = wni8.reshape(N * K, 8)

    wpad = jnp.zeros((8, 8), jnp.float32).at[:3].set(wn_W0)
    row = lambda v: v.reshape(1, -1)

    new_feat = pl.pallas_call(
        _dense_kernel,
        grid=(N // P3,),
        in_specs=[
            pl.BlockSpec((PK, CIN), lambda i: (i, 0)),
            pl.BlockSpec((PK, 8), lambda i: (i, 0)),
            pl.BlockSpec((P3, CIN), lambda i: (i, 0)),
            pl.BlockSpec((CIN, C4), lambda i: (0, 0)),
            pl.BlockSpec((1, C4), lambda i: (0, 0)),
            pl.BlockSpec((1, C4), lambda i: (0, 0)),
            pl.BlockSpec((8, 8), lambda i: (0, 0)),
            pl.BlockSpec((1, 8), lambda i: (0, 0)),
            pl.BlockSpec((1, 8), lambda i: (0, 0)),
            pl.BlockSpec((1, 8), lambda i: (0, 0)),
            pl.BlockSpec((8, 8), lambda i: (0, 0)),
            pl.BlockSpec((1, 8), lambda i: (0, 0)),
            pl.BlockSpec((1, 8), lambda i: (0, 0)),
            pl.BlockSpec((1, 8), lambda i: (0, 0)),
            pl.BlockSpec((8, CMID), lambda i: (0, 0)),
            pl.BlockSpec((1, CMID), lambda i: (0, 0)),
            pl.BlockSpec((1, CMID), lambda i: (0, 0)),
            pl.BlockSpec((1, CMID), lambda i: (0, 0)),
            pl.BlockSpec((C4 * CMID, 64), lambda i: (0, 0)),
            pl.BlockSpec((1, 64), lambda i: (0, 0)),
            pl.BlockSpec((1, 64), lambda i: (0, 0)),
            pl.BlockSpec((1, 64), lambda i: (0, 0)),
            pl.BlockSpec((64, CIN), lambda i: (0, 0)),
            pl.BlockSpec((1, CIN), lambda i: (0, 0)),
            pl.BlockSpec((1, CIN), lambda i: (0, 0)),
        ],
        out_specs=pl.BlockSpec((P3, CIN), lambda i: (i, 0)),
        out_shape=jax.ShapeDtypeStruct((N, CIN), jnp.float32),
    )(gf, wni8, feats,
      u1_W, row(u1_g), row(u1_be),
      wpad, row(wn_b0), row(wn_g0), row(wn_be0),
      wn_W1, row(wn_b1), row(wn_g1), row(wn_be1),
      wn_W2, row(wn_b2), row(wn_g2), row(wn_be2),
      lin_W, row(lin_b), row(lin_g), row(lin_be),
      u2_W, row(u2_g), row(u2_be))

    wni = wni8.reshape(N, K, 8)[:, :, :3]
    return new_feat[None], wni[None]


# speedup vs baseline: 12.6177x; 1.4581x over previous
import functools

import jax
import jax.numpy as jnp
from jax import lax
from jax.experimental import pallas as pl
from jax.experimental.pallas import tpu as pltpu
from jax.experimental.pallas import tpu_sc as plsc

N = 10000
K = 32
CIN = 128
C4 = 32
CMID = 16
CHUNK = 80
NCHUNK = (N * K) // CHUNK
NWORK = 32
CPW = NCHUNK // NWORK
GRP = 5
NGRP = CPW // GRP
P3 = 400
PK = P3 * K

_BN_INV = (1.0 + 1e-5) ** -0.5


def _bn(x, g, b):
    return g * (x * _BN_INV) + b


def _lrelu(x):
    return jnp.where(x > 0, x, 0.1 * x)


def _sc_gather_kernel(featsb_hbm, xyzt_hbm, nei_hbm, gf_hbm, wni_hbm,
                      idx_v, gf_bufs, wni_bufs, xyz_v,
                      gsem, o0, o1, o2, o3, o4):
    osem = [o0, o1, o2, o3, o4]
    wid = lax.axis_index("s") * 2 + lax.axis_index("c")
    base = wid * CPW
    pltpu.sync_copy(nei_hbm.at[wid], idx_v)
    pltpu.sync_copy(xyzt_hbm, xyz_v)

    def drain_out(b):
        pltpu.make_async_copy(gf_hbm.at[pl.ds(0, CHUNK)], gf_bufs.at[b],
                              osem[b]).wait()

    lanes = lax.iota(jnp.int32, 16)
    zero16 = jnp.zeros((16,), jnp.float32)
    for q in range(GRP * CHUNK * 8 // 16):
        wni_bufs[pl.ds(q * 16, 16)] = zero16

    def group(it, carry):
        c0 = base + it * GRP
        handles = []
        for b in range(GRP):

            @pl.when(it > 0)
            def _():
                drain_out(b)

            handles.append(
                pltpu.async_copy(featsb_hbm.at[idx_v.at[it * GRP + b, 0]],
                                 gf_bufs.at[b], gsem))
        for b in range(GRP):
            ci = it * GRP + b
            ebase = (base + ci) * CHUNK
            for r in range(CHUNK // 16):
                iv = idx_v[ci, 0, pl.ds(r * 16, 16)]
                j = r * 16 + lanes
                p = lax.shift_right_logical(ebase + j, 5)
                col = b * CHUNK * 8 + 8 * j
                nx = plsc.load_gather(xyz_v, [iv])
                cx = plsc.load_gather(xyz_v, [p])
                plsc.store_scatter(wni_bufs, [col], nx - cx)
                ny = plsc.load_gather(xyz_v, [iv + N])
                cy = plsc.load_gather(xyz_v, [p + N])
                plsc.store_scatter(wni_bufs, [col + 1], ny - cy)
                nz = plsc.load_gather(xyz_v, [iv + 2 * N])
                cz = plsc.load_gather(xyz_v, [p + 2 * N])
                plsc.store_scatter(wni_bufs, [col + 2], nz - cz)
        for h in handles:
            h.wait()
        for b in range(GRP):
            pltpu.async_copy(gf_bufs.at[b],
                             gf_hbm.at[pl.ds((c0 + b) * CHUNK, CHUNK)],
                             osem[b])
        pltpu.sync_copy(
            wni_bufs, wni_hbm.at[pl.ds(c0 * CHUNK * 8, GRP * CHUNK * 8)])
        return carry

    lax.fori_loop(0, NGRP, group, 0)
    for b in range(GRP):
        drain_out(b)


def _dense_kernel(gf_ref, loc_ref, feats_ref,
                  u1w_ref, u1b_ref,
                  w0_ref, b0_ref,
                  w1_ref, b1_ref,
                  w2_ref, b2_ref,
                  lw_ref, lb_ref,
                  u2w_ref, u2b_ref,
                  nf_ref, wni_ref):
    fx = jnp.dot(gf_ref[...], u1w_ref[...],
                 preferred_element_type=jnp.float32)
    fx = _lrelu(fx + u1b_ref[...]).astype(jnp.bfloat16)

    locf = loc_ref[...]
    h = jnp.dot(locf, w0_ref[...], preferred_element_type=jnp.float32)
    h = jax.nn.relu(h + b0_ref[...])
    h = jnp.dot(h, w1_ref[...], preferred_element_type=jnp.float32)
    h = jax.nn.relu(h + b1_ref[...])
    h = jnp.dot(h, w2_ref[...], preferred_element_type=jnp.float32)
    w = jax.nn.relu(h + b2_ref[...]).astype(jnp.bfloat16)

    gf3 = fx.reshape(P3, K, C4)
    w3 = w.reshape(P3, K, CMID)
    nf = lax.dot_general(gf3, w3, (((1,), (1,)), ((0,), (0,))),
                         preferred_element_type=jnp.float32)
    nf = nf.astype(jnp.bfloat16).reshape(P3, C4 * CMID)

    wni_ref[...] = locf.reshape(P3, K, 8)[:, :, :3]

    y = jnp.dot(nf, lw_ref[...], preferred_element_type=jnp.float32)
    y = jax.nn.relu(y + lb_ref[...])
    y = jnp.dot(y, u2w_ref[...], preferred_element_type=jnp.float32)
    y = y + u2b_ref[...]
    nf_ref[...] = _lrelu(y + feats_ref[...])


def kernel(dense_xyz, dense_feats, inv_neighbors, inv_k, inv_idx, nei_inds,
           u1_W, u1_g, u1_be,
           wn_W0, wn_b0, wn_g0, wn_be0,
           wn_W1, wn_b1, wn_g1, wn_be1,
           wn_W2, wn_b2, wn_g2, wn_be2,
           lin_W, lin_b, lin_g, lin_be,
           u2_W, u2_g, u2_be):
    feats = dense_feats[0]
    xyzt = jnp.transpose(dense_xyz[0]).reshape(3 * N)
    nei = nei_inds.reshape(NWORK, CPW, 1, CHUNK)

    gather = functools.partial(
        pl.kernel,
        mesh=plsc.VectorSubcoreMesh(core_axis_name="c", subcore_axis_name="s"),
        compiler_params=pltpu.CompilerParams(needs_layout_passes=False),
        out_type=[
            jax.ShapeDtypeStruct((N * K, CIN), jnp.float32),
            jax.ShapeDtypeStruct((N * K * 8,), jnp.float32),
        ],
        scratch_types=[
            pltpu.VMEM((CPW, 1, CHUNK), jnp.int32),
            pltpu.VMEM((GRP, CHUNK, CIN), jnp.float32),
            pltpu.VMEM((GRP * CHUNK * 8,), jnp.float32),
            pltpu.VMEM((3 * N,), jnp.float32),
        ] + [pltpu.SemaphoreType.DMA] * 6,
    )(_sc_gather_kernel)
    gf, wni8 = gather(feats, xyzt, nei)
    wni2d = wni8.reshape(N * K, 8)

    row = lambda v: v.reshape(1, -1)
    inv = jnp.float32(_BN_INV)
    u1w_f = u1_W * (u1_g * inv)
    wpad = jnp.zeros((8, 8), jnp.float32).at[:3].set(wn_W0)
    w0_f = wpad * (wn_g0 * inv)
    b0_f = wn_b0 * (wn_g0 * inv) + wn_be0
    w1_f = wn_W1 * (wn_g1 * inv)
    b1_f = wn_b1 * (wn_g1 * inv) + wn_be1
    w2_f = wn_W2 * (wn_g2 * inv)
    b2_f = wn_b2 * (wn_g2 * inv) + wn_be2
    lw_f = (lin_W * (lin_g * inv)).astype(jnp.bfloat16)
    lb_f = lin_b * (lin_g * inv) + lin_be
    u2w_f = u2_W * (u2_g * inv)
    u2b_f = u2_be

    new_feat, wni3 = pl.pallas_call(
        _dense_kernel,
        grid=(N // P3,),
        in_specs=[
            pl.BlockSpec((PK, CIN), lambda i: (i, 0)),
            pl.BlockSpec((PK, 8), lambda i: (i, 0)),
            pl.BlockSpec((P3, CIN), lambda i: (i, 0)),
            pl.BlockSpec((CIN, C4), lambda i: (0, 0)),
            pl.BlockSpec((1, C4), lambda i: (0, 0)),
            pl.BlockSpec((8, 8), lambda i: (0, 0)),
            pl.BlockSpec((1, 8), lambda i: (0, 0)),
            pl.BlockSpec((8, 8), lambda i: (0, 0)),
            pl.BlockSpec((1, 8), lambda i: (0, 0)),
            pl.BlockSpec((8, CMID), lambda i: (0, 0)),
            pl.BlockSpec((1, CMID), lambda i: (0, 0)),
            pl.BlockSpec((C4 * CMID, 64), lambda i: (0, 0)),
            pl.BlockSpec((1, 64), lambda i: (0, 0)),
            pl.BlockSpec((64, CIN), lambda i: (0, 0)),
            pl.BlockSpec((1, CIN), lambda i: (0, 0)),
        ],
        out_specs=[
            pl.BlockSpec((P3, CIN), lambda i: (i, 0)),
            pl.BlockSpec((P3, K, 3), lambda i: (i, 0, 0)),
        ],
        out_shape=[
            jax.ShapeDtypeStruct((N, CIN), jnp.float32),
            jax.ShapeDtypeStruct((N, K, 3), jnp.float32),
        ],
    )(gf, wni2d, feats,
      u1w_f, row(u1_be),
      w0_f, row(b0_f),
      w1_f, row(b1_f),
      w2_f, row(b2_f),
      lw_f, row(lb_f),
      u2w_f, row(u2b_f))

    return new_feat[None], wni3[None]
